# baseline (speedup 1.0000x reference)
import jax
import jax.numpy as jnp
from jax.experimental import pallas as pl
from jax.experimental.pallas import tpu as pltpu

NUM_TOKENS = 16384
INPUT_DIM = 2048
NUM_EXPERTS = 64
CH = 512
NCH = NUM_TOKENS // CH
NBUF = 8


def _gate_kernel(x_hbm, wt_ref, b_ref, ids_ref, gs_ref, idx_ref, xbuf, sem):
    def issue(c):
        pltpu.make_async_copy(
            x_hbm.at[pl.ds(c * CH, CH), :], xbuf.at[c % NBUF], sem.at[c % NBUF]
        ).start()

    for c in range(NBUF):
        issue(c)
    for c in range(NCH):
        pltpu.make_async_copy(
            x_hbm.at[pl.ds(c * CH, CH), :], xbuf.at[c % NBUF], sem.at[c % NBUF]
        ).wait()
        logits = jnp.dot(xbuf[c % NBUF], wt_ref[...],
                         preferred_element_type=jnp.float32) + b_ref[...]
        if c + NBUF < NCH:
            issue(c + NBUF)
        ids = jnp.broadcast_to(ids_ref[...], logits.shape)
        big = jnp.float32(NUM_EXPERTS)
        v1 = jnp.max(logits, axis=1, keepdims=True)
        i1 = jnp.min(jnp.where(logits == v1, ids, big), axis=1, keepdims=True)
        masked = jnp.where(ids == i1, -jnp.inf, logits)
        v2 = jnp.max(masked, axis=1, keepdims=True)
        i2 = jnp.min(jnp.where(masked == v2, ids, big), axis=1, keepdims=True)
        e2 = jnp.exp(v2 - v1)
        denom = 1.0 + e2
        rows = pl.ds(c * CH, CH)
        gs_ref[rows, :] = jnp.concatenate([1.0 / denom, e2 / denom], axis=1)
        idx_ref[rows, :] = jnp.concatenate([i1, i2], axis=1).astype(jnp.int32)


def kernel(x, W, b):
    wt = W.T
    b2 = b.reshape(1, NUM_EXPERTS)
    ids_row = jnp.arange(NUM_EXPERTS, dtype=jnp.float32).reshape(1, NUM_EXPERTS)
    gs, idx = pl.pallas_call(
        _gate_kernel,
        in_specs=[
            pl.BlockSpec(memory_space=pltpu.MemorySpace.HBM),
            pl.BlockSpec(memory_space=pltpu.MemorySpace.VMEM),
            pl.BlockSpec(memory_space=pltpu.MemorySpace.VMEM),
            pl.BlockSpec(memory_space=pltpu.MemorySpace.VMEM),
        ],
        out_specs=[
            pl.BlockSpec(memory_space=pltpu.MemorySpace.VMEM),
            pl.BlockSpec(memory_space=pltpu.MemorySpace.VMEM),
        ],
        out_shape=[
            jax.ShapeDtypeStruct((NUM_TOKENS, 2), jnp.float32),
            jax.ShapeDtypeStruct((NUM_TOKENS, 2), jnp.int32),
        ],
        scratch_shapes=[
            pltpu.MemorySpace.VMEM((NBUF, CH, INPUT_DIM), jnp.float32),
            pltpu.SemaphoreType.DMA((NBUF,)),
        ],
    )(x, wt, b2, ids_row)
    return gs, idx


# speedup vs baseline: 1.1950x; 1.1950x over previous
import jax
import jax.numpy as jnp
from jax.experimental import pallas as pl

NUM_TOKENS = 16384
INPUT_DIM = 2048
NUM_EXPERTS = 64
BT = 2048
KH = INPUT_DIM // 2


def _gate_kernel(xl_ref, xr_ref, wtl_ref, wtr_ref, b_ref, ids_ref,
                 gs_ref, idx_ref):
    logits = (jnp.dot(xl_ref[...], wtl_ref[...],
                      preferred_element_type=jnp.float32)
              + jnp.dot(xr_ref[...], wtr_ref[...],
                        preferred_element_type=jnp.float32)
              + b_ref[...])
    ids = jnp.broadcast_to(ids_ref[...], logits.shape)
    big = jnp.float32(NUM_EXPERTS)
    v1 = jnp.max(logits, axis=1, keepdims=True)
    i1 = jnp.min(jnp.where(logits == v1, ids, big), axis=1, keepdims=True)
    masked = jnp.where(ids == i1, -jnp.inf, logits)
    v2 = jnp.max(masked, axis=1, keepdims=True)
    i2 = jnp.min(jnp.where(masked == v2, ids, big), axis=1, keepdims=True)
    e2 = jnp.exp(v2 - v1)
    denom = 1.0 + e2
    gs_ref[...] = jnp.concatenate([1.0 / denom, e2 / denom], axis=1)
    idx_ref[...] = jnp.concatenate([i1, i2], axis=1).astype(jnp.int32)


def kernel(x, W, b):
    wt = W.T
    b2 = b.reshape(1, NUM_EXPERTS)
    ids_row = jnp.arange(NUM_EXPERTS, dtype=jnp.float32).reshape(1, NUM_EXPERTS)
    grid = (NUM_TOKENS // BT,)
    gs, idx = pl.pallas_call(
        _gate_kernel,
        grid=grid,
        in_specs=[
            pl.BlockSpec((BT, KH), lambda i: (i, 0)),
            pl.BlockSpec((BT, KH), lambda i: (i, 1)),
            pl.BlockSpec((KH, NUM_EXPERTS), lambda i: (0, 0)),
            pl.BlockSpec((KH, NUM_EXPERTS), lambda i: (1, 0)),
            pl.BlockSpec((1, NUM_EXPERTS), lambda i: (0, 0)),
            pl.BlockSpec((1, NUM_EXPERTS), lambda i: (0, 0)),
        ],
        out_specs=[
            pl.BlockSpec((BT, 2), lambda i: (i, 0)),
            pl.BlockSpec((BT, 2), lambda i: (i, 0)),
        ],
        out_shape=[
            jax.ShapeDtypeStruct((NUM_TOKENS, 2), jnp.float32),
            jax.ShapeDtypeStruct((NUM_TOKENS, 2), jnp.int32),
        ],
    )(x, x, wt, wt, b2, ids_row)
    return gs, idx
